# Initial kernel scaffold; baseline (speedup 1.0000x reference)
import functools

import jax
import jax.numpy as jnp
from jax import lax
from jax.experimental import pallas as pl
from jax.experimental.pallas import tpu as pltpu
from jax.experimental.pallas import tpu_sc as plsc

_VOCAB = 100000
_DIM = 128
_B = 1024
_S = 200
_N = _B * _S
_NW = 32
_ROWS_PER_W = _N // _NW
_CHUNK = 256
_NCHUNK = _ROWS_PER_W // _CHUNK
_NGROUP = _CHUNK // 16
_EPS = 1e-12


def _rsqrt(v):
    i = plsc.bitcast(v, jnp.int32)
    i = 0x5F3759DF - lax.shift_right_arithmetic(i, 1)
    y = plsc.bitcast(i, jnp.float32)
    for _ in range(3):
        y = y * (1.5 - 0.5 * v * y * y)
    return y


def _make_kernel():
    mesh = plsc.VectorSubcoreMesh(core_axis_name="c", subcore_axis_name="s")

    @functools.partial(
        pl.kernel,
        mesh=mesh,
        out_type=jax.ShapeDtypeStruct((_N, _DIM), jnp.float32),
        scratch_types=[
            pltpu.VMEM((_ROWS_PER_W,), jnp.int32),
            pltpu.VMEM((_S, _DIM), jnp.float32),
            pltpu.VMEM((_DIM,), jnp.float32),
            pltpu.VMEM((_DIM,), jnp.float32),
            pltpu.VMEM((_CHUNK, _DIM), jnp.float32),
            pltpu.SemaphoreType.DMA,
        ],
    )
    def k(ids_hbm, wt_hbm, pos_hbm, gamma_hbm, beta_hbm, out_hbm,
          ids_v, pos_v, gamma_v, beta_v, buf, sem):
        wid = lax.axis_index("s") * 2 + lax.axis_index("c")
        wbase = wid * _ROWS_PER_W

        pltpu.sync_copy(ids_hbm.at[pl.ds(wbase, _ROWS_PER_W)], ids_v)
        pltpu.sync_copy(pos_hbm.at[pl.ds(0, _S)], pos_v)
        pltpu.sync_copy(gamma_hbm, gamma_v)
        pltpu.sync_copy(beta_hbm, beta_v)

        iota = lax.iota(jnp.int32, 16)

        def do_group(t, g):
            c0 = t * 16
            ids_vec = ids_v[pl.ds(g * _CHUNK + c0, 16)]
            maskf = jnp.where(ids_vec == 0, 0.0, 1.0).astype(jnp.float32)
            rvec = c0 + iota
            pvec = lax.rem(wbase + g * _CHUNK + c0 + iota, _S)

            def pass1(j, carry):
                s, q = carry
                jvec = jnp.full((16,), 0, jnp.int32) + j
                w = plsc.load_gather(buf, [rvec, jvec])
                p = plsc.load_gather(pos_v, [pvec, jvec])
                x = w * maskf + p
                plsc.store_scatter(buf, [rvec, jvec], x)
                return (s + x, q + x * x)

            zero = jnp.zeros((16,), jnp.float32)
            ssum, ssq = lax.fori_loop(0, _DIM, pass1, (zero, zero))

            mean = ssum * (1.0 / _DIM)
            var = ssq * (1.0 / _DIM) - mean * mean
            var = jnp.maximum(var, 0.0)
            rstd = _rsqrt(var + _EPS)
            m2 = mean * rstd

            def pass2(j, _):
                jvec = jnp.full((16,), 0, jnp.int32) + j
                x = plsc.load_gather(buf, [rvec, jvec])
                out = (x * rstd - m2) * gamma_v[j] + beta_v[j]
                plsc.store_scatter(buf, [rvec, jvec], out)
                return 0

            lax.fori_loop(0, _DIM, pass2, 0)
            return 0

        def do_chunk(g, _):
            idx = ids_v.at[pl.ds(g * _CHUNK, _CHUNK)]
            pltpu.async_copy(wt_hbm.at[idx], buf, sem).wait()
            lax.fori_loop(0, _NGROUP, lambda t, c: do_group(t, g), 0)
            pltpu.sync_copy(buf, out_hbm.at[pl.ds(wbase + g * _CHUNK, _CHUNK)])
            return 0

        lax.fori_loop(0, _NCHUNK, do_chunk, 0)

    return k


_kernel_fn = _make_kernel()


@jax.jit
def kernel(input_ids, word_table, pos_table, gamma, beta):
    ids_flat = input_ids.reshape(_N)
    out = _kernel_fn(ids_flat, word_table, pos_table, gamma, beta)
    return out.reshape(_B, _S, _DIM)


# speedup vs baseline: 1.8893x; 1.8893x over previous
import functools

import jax
import jax.numpy as jnp
from jax import lax
from jax.experimental import pallas as pl
from jax.experimental.pallas import tpu as pltpu
from jax.experimental.pallas import tpu_sc as plsc

_DIM = 128
_B = 1024
_S = 200
_N = _B * _S
_NW = 32
_ROWS_PER_W = _N // _NW
_CHUNK = 256
_NCHUNK = _ROWS_PER_W // _CHUNK
_NCB = _DIM // 16
_EPS = 1e-12


def _rsqrt(v):
    i = plsc.bitcast(v, jnp.int32)
    i = 0x5F3759DF - lax.shift_right_arithmetic(i, 1)
    y = plsc.bitcast(i, jnp.float32)
    for _ in range(3):
        y = y * (1.5 - 0.5 * v * y * y)
    return y


def _make_kernel():
    mesh = plsc.VectorSubcoreMesh(core_axis_name="c", subcore_axis_name="s")

    @functools.partial(
        pl.kernel,
        mesh=mesh,
        out_type=jax.ShapeDtypeStruct((_N, _DIM), jnp.float32),
        scratch_types=[
            pltpu.VMEM((_ROWS_PER_W,), jnp.int32),
            pltpu.VMEM((_S, _DIM), jnp.float32),
            pltpu.VMEM((_DIM,), jnp.float32),
            pltpu.VMEM((_DIM,), jnp.float32),
            pltpu.VMEM((_CHUNK, _DIM), jnp.float32),
            pltpu.SemaphoreType.DMA,
        ],
        compiler_params=pltpu.CompilerParams(needs_layout_passes=False),
    )
    def k(ids_hbm, wt_hbm, pos_hbm, gamma_hbm, beta_hbm, out_hbm,
          ids_v, pos_v, gamma_v, beta_v, buf, sem):
        wid = lax.axis_index("s") * 2 + lax.axis_index("c")
        wbase = wid * _ROWS_PER_W

        pltpu.sync_copy(ids_hbm.at[pl.ds(wbase, _ROWS_PER_W)], ids_v)
        pltpu.sync_copy(pos_hbm.at[pl.ds(0, _S)], pos_v)
        pltpu.sync_copy(gamma_hbm, gamma_v)
        pltpu.sync_copy(beta_hbm, beta_v)

        gvec = [gamma_v[pl.ds(c * 16, 16)] for c in range(_NCB)]
        bvec = [beta_v[pl.ds(c * 16, 16)] for c in range(_NCB)]

        def do_row(r, g):
            rr = g * _CHUNK + r
            idsb = plsc.load_gather(ids_v, [jnp.full((16,), 0, jnp.int32) + rr])
            maskf = jnp.where(idsb == 0, 0.0, 1.0).astype(jnp.float32)
            pr = lax.rem(wbase + rr, _S)

            x = []
            s = None
            q = None
            for c in range(_NCB):
                w = buf[r, pl.ds(c * 16, 16)]
                p = pos_v[pr, pl.ds(c * 16, 16)]
                xc = w * maskf + p
                x.append(xc)
                s = xc if s is None else s + xc
                q = xc * xc if q is None else q + xc * xc

            tot = jnp.sum(s)
            totq = jnp.sum(q)
            mean = jnp.full((16,), 0.0, jnp.float32) + tot * (1.0 / _DIM)
            var = jnp.full((16,), 0.0, jnp.float32) + totq * (1.0 / _DIM)
            var = jnp.maximum(var - mean * mean, 0.0)
            rstd = _rsqrt(var + _EPS)
            m2 = mean * rstd

            for c in range(_NCB):
                buf[r, pl.ds(c * 16, 16)] = (x[c] * rstd - m2) * gvec[c] + bvec[c]
            return g

        def do_chunk(g, _):
            idx = ids_v.at[pl.ds(g * _CHUNK, _CHUNK)]
            pltpu.async_copy(wt_hbm.at[idx], buf, sem).wait()
            lax.fori_loop(0, _CHUNK, do_row, g)
            pltpu.sync_copy(buf, out_hbm.at[pl.ds(wbase + g * _CHUNK, _CHUNK)])
            return 0

        lax.fori_loop(0, _NCHUNK, do_chunk, 0)

    return k


_kernel_fn = _make_kernel()


@jax.jit
def kernel(input_ids, word_table, pos_table, gamma, beta):
    ids_flat = input_ids.reshape(_N)
    out = _kernel_fn(ids_flat, word_table, pos_table, gamma, beta)
    return out.reshape(_B, _S, _DIM)
